# baseline (speedup 1.0000x reference)
import math

import jax
import jax.numpy as jnp
import numpy as np
from jax import lax
from jax.experimental import pallas as pl
from jax.experimental.pallas import tpu as pltpu

H, W = 512, 1024
EPS = 1e-05
B, F = 64, 50
N = H * W
SUB, LANE = 8, 128


def _row_tables():
    thetas = np.linspace(0.5, H - 0.5, num=H) * math.pi / H
    weight = 1.0 / np.sin(thetas)
    residual = weight % 2
    mask = residual >= 1
    residual[mask] -= 1
    residual[~mask] += 1
    n_ones = (weight - residual).astype(np.int32)
    edge_values = ((weight - n_ones) / 2).astype(np.float32)
    lengths = n_ones + 2
    return lengths.astype(np.int32), edge_values


_LEN_NP, _EV_NP = _row_tables()


def _nss_kernel(rows_ref, lefts_ref, widths_ref, prevs_ref, evs_ref, lasts_ref,
                a_ref, out_ref, buf_ref):
    a = a_ref[0]
    s1 = jnp.sum(a)
    s2 = jnp.sum(a * a)

    pos = (lax.broadcasted_iota(jnp.int32, (SUB, LANE), 0) * LANE
           + lax.broadcasted_iota(jnp.int32, (SUB, LANE), 1))

    def step(f, carry):
        sdot_v, sfm_v = carry
        y = rows_ref[0, 0, f]
        left = lefts_ref[0, 0, f]
        kw = widths_ref[0, 0, f]
        ev = evs_ref[0, 0, f]
        p = prevs_ref[0, 0, f]
        il = lasts_ref[0, 0, f]
        start = buf_ref[jnp.maximum(p, 0)]
        start = jnp.where(p >= 0, start, 0.0)
        off = (pos - left) & (W - 1)
        covered = off < kw
        val = jnp.where((off == 0) | (off == kw - 1), ev, 1.0)
        new = jnp.where(covered, val, start)
        edge = (y == 0) | (y == H - 1)
        new = jnp.where(edge, 1.0, new)
        buf_ref[f] = new
        arow = a_ref[0, y]
        keep = il * new
        return sdot_v + keep * arow, sfm_v + keep

    zero = jnp.zeros((SUB, LANE), jnp.float32)
    sdot_v, sfm_v = lax.fori_loop(0, F, step, (zero, zero), unroll=False)
    sdot = jnp.sum(sdot_v)
    sfm = jnp.sum(sfm_v)

    mean = s1 / N
    var = (s2 - s1 * s1 / N) / (N - 1)
    std = jnp.sqrt(var)
    denom = std + jnp.where(std < EPS, EPS, 0.0)
    contrib = (sdot - mean * sfm) / (denom * (F * B))

    b = pl.program_id(0)

    @pl.when(b == 0)
    def _():
        out_ref[0, 0] = contrib

    @pl.when(b > 0)
    def _():
        out_ref[0, 0] += contrib


def kernel(y_pred, y_gt):
    lengths = jnp.asarray(_LEN_NP)
    evs_tab = jnp.asarray(_EV_NP)

    x_idx = jnp.rint(y_gt[:, :, 0] * (W - 1)).astype(jnp.int32)
    y_idx = jnp.rint(y_gt[:, :, 1] * (H - 1)).astype(jnp.int32)
    kw = lengths[y_idx]
    ev = evs_tab[y_idx]
    left = x_idx - kw // 2

    jj = jnp.arange(F, dtype=jnp.int32)
    same = y_idx[:, :, None] == y_idx[:, None, :]
    before = jj[None, None, :] < jj[None, :, None]
    after = jj[None, None, :] > jj[None, :, None]
    prev = jnp.max(jnp.where(same & before, jj[None, None, :], -1), axis=2)
    is_last = jnp.logical_not(jnp.any(same & after, axis=2))
    il = is_last.astype(jnp.float32)

    a = y_pred.reshape(B, H, SUB, LANE)

    smem_i = pl.BlockSpec((1, 1, F), lambda b: (b, 0, 0), memory_space=pltpu.SMEM)

    out = pl.pallas_call(
        _nss_kernel,
        grid=(B,),
        in_specs=[
            smem_i, smem_i, smem_i, smem_i, smem_i, smem_i,
            pl.BlockSpec((1, H, SUB, LANE), lambda b: (b, 0, 0, 0)),
        ],
        out_specs=pl.BlockSpec((1, 1), lambda b: (0, 0), memory_space=pltpu.SMEM),
        out_shape=jax.ShapeDtypeStruct((1, 1), jnp.float32),
        scratch_shapes=[pltpu.VMEM((F, SUB, LANE), jnp.float32)],
    )(
        y_idx.reshape(B, 1, F),
        left.reshape(B, 1, F),
        kw.reshape(B, 1, F),
        prev.reshape(B, 1, F),
        ev.reshape(B, 1, F),
        il.reshape(B, 1, F),
        a,
    )
    return out[0, 0]


# speedup vs baseline: 1.2613x; 1.2613x over previous
import math

import jax
import jax.numpy as jnp
import numpy as np
from jax import lax
from jax.experimental import pallas as pl
from jax.experimental.pallas import tpu as pltpu

H, W = 512, 1024
EPS = 1e-05
B, F = 64, 50
N = H * W


def _row_tables():
    thetas = np.linspace(0.5, H - 0.5, num=H) * math.pi / H
    weight = 1.0 / np.sin(thetas)
    residual = weight % 2
    mask = residual >= 1
    residual[mask] -= 1
    residual[~mask] += 1
    n_ones = (weight - residual).astype(np.int32)
    edge_values = ((weight - n_ones) / 2).astype(np.float32)
    lengths = n_ones + 2
    return lengths.astype(np.int32), edge_values


_LEN_NP, _EV_NP = _row_tables()


def _nss_kernel(rows_ref, lefts_ref, widths_ref, prevs_ref, evs_ref, lasts_ref,
                a_ref, out_ref, buf_ref):
    ar = a_ref[0, 0].reshape(H // 8, 8, W)
    s1p = jnp.sum(ar, axis=0)
    s2p = jnp.sum(ar * ar, axis=0)
    s1 = jnp.sum(s1p)
    s2 = jnp.sum(s2p)

    pos = lax.broadcasted_iota(jnp.int32, (1, W), 1)

    def step(f, carry):
        sdot_v, sfm_v = carry
        y = rows_ref[0, 0, f]
        left = lefts_ref[0, 0, f]
        kw = widths_ref[0, 0, f]
        ev = evs_ref[0, 0, f]
        p = prevs_ref[0, 0, f]
        il = lasts_ref[0, 0, f]
        start = buf_ref[pl.ds(jnp.maximum(p, 0), 1), :]
        start = jnp.where(p >= 0, start, 0.0)
        off = (pos - left) & (W - 1)
        covered = off < kw
        val = jnp.where((off == 0) | (off == kw - 1), ev, 1.0)
        new = jnp.where(covered, val, start)
        edge = (y == 0) | (y == H - 1)
        new = jnp.where(edge, 1.0, new)
        buf_ref[pl.ds(f, 1), :] = new
        arow = a_ref[0, 0, pl.ds(y, 1), :]
        keep = il * new
        return sdot_v + keep * arow, sfm_v + keep

    zero = jnp.zeros((1, W), jnp.float32)
    sdot_v, sfm_v = lax.fori_loop(0, F, step, (zero, zero), unroll=False)
    sdot = jnp.sum(sdot_v)
    sfm = jnp.sum(sfm_v)

    mean = s1 / N
    var = (s2 - s1 * s1 / N) / (N - 1)
    std = jnp.sqrt(var)
    denom = std + jnp.where(std < EPS, EPS, 0.0)
    contrib = (sdot - mean * sfm) / (denom * (F * B))

    b = pl.program_id(0)

    @pl.when(b == 0)
    def _():
        out_ref[0, 0] = contrib

    @pl.when(b > 0)
    def _():
        out_ref[0, 0] += contrib


def kernel(y_pred, y_gt):
    lengths = jnp.asarray(_LEN_NP)
    evs_tab = jnp.asarray(_EV_NP)

    x_idx = jnp.rint(y_gt[:, :, 0] * (W - 1)).astype(jnp.int32)
    y_idx = jnp.rint(y_gt[:, :, 1] * (H - 1)).astype(jnp.int32)
    kw = lengths[y_idx]
    ev = evs_tab[y_idx]
    left = x_idx - kw // 2

    jj = jnp.arange(F, dtype=jnp.int32)
    same = y_idx[:, :, None] == y_idx[:, None, :]
    before = jj[None, None, :] < jj[None, :, None]
    after = jj[None, None, :] > jj[None, :, None]
    prev = jnp.max(jnp.where(same & before, jj[None, None, :], -1), axis=2)
    is_last = jnp.logical_not(jnp.any(same & after, axis=2))
    il = is_last.astype(jnp.float32)

    smem_i = pl.BlockSpec((1, 1, F), lambda b: (b, 0, 0), memory_space=pltpu.SMEM)

    out = pl.pallas_call(
        _nss_kernel,
        grid=(B,),
        in_specs=[
            smem_i, smem_i, smem_i, smem_i, smem_i, smem_i,
            pl.BlockSpec((1, 1, H, W), lambda b: (b, 0, 0, 0)),
        ],
        out_specs=pl.BlockSpec((1, 1), lambda b: (0, 0), memory_space=pltpu.SMEM),
        out_shape=jax.ShapeDtypeStruct((1, 1), jnp.float32),
        scratch_shapes=[pltpu.VMEM((F, W), jnp.float32)],
    )(
        y_idx.reshape(B, 1, F),
        left.reshape(B, 1, F),
        kw.reshape(B, 1, F),
        prev.reshape(B, 1, F),
        ev.reshape(B, 1, F),
        il.reshape(B, 1, F),
        y_pred,
    )
    return out[0, 0]


# speedup vs baseline: 1.4792x; 1.1728x over previous
import math

import jax
import jax.numpy as jnp
import numpy as np
from jax import lax
from jax.experimental import pallas as pl
from jax.experimental.pallas import tpu as pltpu

H, W = 512, 1024
EPS = 1e-05
B, F = 64, 50
N = H * W


def _row_tables():
    thetas = np.linspace(0.5, H - 0.5, num=H) * math.pi / H
    weight = 1.0 / np.sin(thetas)
    residual = weight % 2
    mask = residual >= 1
    residual[mask] -= 1
    residual[~mask] += 1
    n_ones = (weight - residual).astype(np.int32)
    edge_values = ((weight - n_ones) / 2).astype(np.float32)
    lengths = n_ones + 2
    return lengths.astype(np.int32), edge_values


_LEN_NP, _EV_NP = _row_tables()


def _nss_kernel(rows_ref, lefts_ref, widths_ref, prevs_ref, evs_ref, lasts_ref,
                a_ref, out_ref, buf_ref):
    ar = a_ref[0, 0].reshape(H // 8, 8, W)
    s1p = jnp.sum(ar, axis=0)
    s2p = jnp.sum(ar * ar, axis=0)
    s1 = jnp.sum(s1p)
    s2 = jnp.sum(s2p)

    pos = lax.broadcasted_iota(jnp.int32, (1, W), 1)

    def step(f, carry):
        sdot_v, sfm_v = carry
        y = rows_ref[0, 0, f]
        left = lefts_ref[0, 0, f]
        kw = widths_ref[0, 0, f]
        ev = evs_ref[0, 0, f]
        p = prevs_ref[0, 0, f]
        il = lasts_ref[0, 0, f]
        start = buf_ref[pl.ds(jnp.maximum(p, 0), 1), :]
        start = jnp.where(p >= 0, start, 0.0)
        off = (pos - left) & (W - 1)
        covered = off < kw
        val = jnp.where((off == 0) | (off == kw - 1), ev, 1.0)
        new = jnp.where(covered, val, start)
        edge = (y == 0) | (y == H - 1)
        new = jnp.where(edge, 1.0, new)
        buf_ref[pl.ds(f, 1), :] = new
        arow = a_ref[0, 0, pl.ds(y, 1), :]
        keep = il * new
        return sdot_v + keep * arow, sfm_v + keep

    zero = jnp.zeros((1, W), jnp.float32)
    sdot_v, sfm_v = lax.fori_loop(0, F, step, (zero, zero), unroll=True)
    sdot = jnp.sum(sdot_v)
    sfm = jnp.sum(sfm_v)

    mean = s1 / N
    var = (s2 - s1 * s1 / N) / (N - 1)
    std = jnp.sqrt(var)
    denom = std + jnp.where(std < EPS, EPS, 0.0)
    contrib = (sdot - mean * sfm) / (denom * (F * B))

    b = pl.program_id(0)

    @pl.when(b == 0)
    def _():
        out_ref[0, 0] = contrib

    @pl.when(b > 0)
    def _():
        out_ref[0, 0] += contrib


def kernel(y_pred, y_gt):
    lengths = jnp.asarray(_LEN_NP)
    evs_tab = jnp.asarray(_EV_NP)

    x_idx = jnp.rint(y_gt[:, :, 0] * (W - 1)).astype(jnp.int32)
    y_idx = jnp.rint(y_gt[:, :, 1] * (H - 1)).astype(jnp.int32)
    kw = lengths[y_idx]
    ev = evs_tab[y_idx]
    left = x_idx - kw // 2

    jj = jnp.arange(F, dtype=jnp.int32)
    same = y_idx[:, :, None] == y_idx[:, None, :]
    before = jj[None, None, :] < jj[None, :, None]
    after = jj[None, None, :] > jj[None, :, None]
    prev = jnp.max(jnp.where(same & before, jj[None, None, :], -1), axis=2)
    is_last = jnp.logical_not(jnp.any(same & after, axis=2))
    il = is_last.astype(jnp.float32)

    smem_i = pl.BlockSpec((1, 1, F), lambda b: (b, 0, 0), memory_space=pltpu.SMEM)

    out = pl.pallas_call(
        _nss_kernel,
        grid=(B,),
        in_specs=[
            smem_i, smem_i, smem_i, smem_i, smem_i, smem_i,
            pl.BlockSpec((1, 1, H, W), lambda b: (b, 0, 0, 0)),
        ],
        out_specs=pl.BlockSpec((1, 1), lambda b: (0, 0), memory_space=pltpu.SMEM),
        out_shape=jax.ShapeDtypeStruct((1, 1), jnp.float32),
        scratch_shapes=[pltpu.VMEM((F, W), jnp.float32)],
    )(
        y_idx.reshape(B, 1, F),
        left.reshape(B, 1, F),
        kw.reshape(B, 1, F),
        prev.reshape(B, 1, F),
        ev.reshape(B, 1, F),
        il.reshape(B, 1, F),
        y_pred,
    )
    return out[0, 0]
